# Initial kernel scaffold; baseline (speedup 1.0000x reference)
import jax
import jax.numpy as jnp
from jax.experimental import pallas as pl


def kernel(node_x, edge_attr, edge_index, schedule, y, W_node, b_node, W_edge, b_edge, W_pinit, b_pinit, Wa, Wb, Wo, bo, Wnu, bnu, W_dec, b_dec):
    raise NotImplementedError("write your pallas kernel here")


# speedup vs baseline: 2.7390x; 2.7390x over previous
import jax
import jax.numpy as jnp
from jax.experimental import pallas as pl
from jax.experimental.pallas import tpu as pltpu

_B = 16
_N = 100
_ND = 16
_D = 128
_L = 3
_S = 8
_E = 12800
_C = 20
_NP = 128


def _prelim_body(ea_ref, nx_ref, We_ref, be_ref, Wp_ref, bp_ref, Wn_ref,
                 bn_ref, ef2_ref, nf0_ref):
    ef = jnp.maximum(
        jnp.dot(ea_ref[...], We_ref[...], preferred_element_type=jnp.float32)
        + be_ref[...], 0.0)
    ef2_ref[...] = jnp.maximum(
        jnp.dot(ef, Wp_ref[...], preferred_element_type=jnp.float32)
        + bp_ref[...], 0.0)
    nf0_ref[...] = jnp.maximum(
        jnp.dot(nx_ref[...], Wn_ref[...], preferred_element_type=jnp.float32)
        + bn_ref[...], 0.0)


def _bdot(lhs, rhs):
    return jax.lax.dot_general(
        lhs, rhs, (((2,), (1,)), ((0,), (0,))),
        preferred_element_type=jnp.float32)


def _main_body(zs_ref, starts_ref, sched_ref,
               ef2_ref, nf0_ref, y_ref,
               WaT_ref, WbT_ref, WoT_ref, Wo_ref, bo_ref, bor_ref,
               Wnu_ref, bnu_ref, WdecT_ref, bdec_ref,
               out_ref,
               pfr_ref, pf4_ref, tri_ref, t3_ref):
    b = pl.program_id(0)

    pfr_ref[...] = jnp.zeros((_N, _NP, _D), jnp.float32)

    def _scatter(e, carry):
        z = zs_ref[e]
        x = z // _NP
        y = z - x * _NP
        row = ef2_ref[pl.ds(e, 1), :]
        pfr_ref[x, pl.ds(y, 1), :] += row
        return carry

    jax.lax.fori_loop(starts_ref[b], starts_ref[b + 1], _scatter, 0)

    pf4_ref[...] = jnp.transpose(pfr_ref[...], (0, 2, 1))

    nfR = nf0_ref[0]

    for l in range(_L):
        last = (l == _L - 1)
        WaB = jnp.broadcast_to(WaT_ref[l][None], (_N, _S, _D))
        WbB = jnp.broadcast_to(WbT_ref[l][None], (_N, _S, _D))
        pf4 = pf4_ref[...]
        AT4 = _bdot(WaB, pf4)
        BT4 = _bdot(WbB, pf4)
        nfT = jnp.transpose(nfR)
        nfA_t = jnp.dot(WaT_ref[l], nfT,
                        preferred_element_type=jnp.float32)
        nfB_t = jnp.dot(WbT_ref[l], nfT,
                        preferred_element_type=jnp.float32)
        if not last:
            t3_ref[...] = jnp.zeros((_S, _NP), jnp.float32)
            t3_ref[:, :_N] = nfB_t
            nfB_row = t3_ref[...]
            for i0 in range(0, _N, 25):
                at_blk = AT4[i0:i0 + 25]
                carry = at_blk[:, :, 0:1] + BT4[0][None]
                for k in range(1, _N):
                    carry = jnp.maximum(
                        carry, at_blk[:, :, k:k + 1] + BT4[k][None])
                for ii in range(25):
                    i = i0 + ii
                    tri_ref[i] = (carry[ii] + nfA_t[:, i:i + 1] + nfB_row)
            WoB = jnp.broadcast_to(WoT_ref[l][None], (_N, _D, _S))
            delta = _bdot(WoB, tri_ref[...])
            pf4_ref[...] = pf4 + jnp.maximum(delta + bo_ref[l][None], 0.0)
            sumj = jnp.sum(pf4_ref[:, :, :_N], axis=2)
            nfR = nfR + jnp.maximum(
                jnp.dot(sumj, Wnu_ref[l],
                        preferred_element_type=jnp.float32)
                + bnu_ref[l], 0.0)
        else:
            tri_ref[...] = AT4
            nfa_rows = jnp.dot(nfR, WaT_ref[l].T,
                               preferred_element_type=jnp.float32)
            nfb_rows = jnp.dot(nfR, WbT_ref[l].T,
                               preferred_element_type=jnp.float32)
            rows = [tri_ref[pl.ds(sched_ref[(b * _C + c) * 2], 1)]
                    for c in range(_C)]
            at3 = jnp.concatenate(rows, axis=0)
            carry3 = at3[:, :, 0:1] + BT4[0][None]
            for k in range(1, _N):
                carry3 = jnp.maximum(
                    carry3, at3[:, :, k:k + 1] + BT4[k][None])
            lane3 = jax.lax.broadcasted_iota(jnp.int32, (1, _S, _NP), 2)
            sub2 = jax.lax.broadcasted_iota(jnp.int32, (_N, _S), 0)
            lane_d = jax.lax.broadcasted_iota(jnp.int32, (1, _D, _NP), 2)
            t3rows = []
            totrow = jnp.zeros((1, _D), jnp.float32)
            for c in range(_C):
                ic = sched_ref[(b * _C + c) * 2]
                jc = sched_ref[(b * _C + c) * 2 + 1]
                val = jnp.sum(jnp.where(lane3 == jc, carry3[c:c + 1], 0.0),
                              axis=2)
                arow = jnp.sum(jnp.where(sub2 == ic, nfa_rows, 0.0),
                               axis=0, keepdims=True)
                brow = jnp.sum(jnp.where(sub2 == jc, nfb_rows, 0.0),
                               axis=0, keepdims=True)
                t3rows.append(val + arow + brow)
                totrow = totrow + jnp.sum(
                    jnp.where(lane_d == jc, pf4_ref[pl.ds(ic, 1)], 0.0),
                    axis=2)
            t3 = jnp.concatenate(t3rows, axis=0)
            delta3 = jnp.maximum(
                jnp.dot(t3, Wo_ref[l], preferred_element_type=jnp.float32)
                + bor_ref[l], 0.0)
            totrow = totrow + jnp.sum(delta3, axis=0, keepdims=True)
            wsum = jnp.sum(WdecT_ref[...], axis=0, keepdims=True)
            pred = (jnp.sum(wsum * totrow) / (_C * _ND)
                    + jnp.sum(bdec_ref[...]) / _ND)
            out_ref[...] = jnp.reshape(jnp.abs(pred - y_ref[0, 0, 0]),
                                       (1, 1, 1))


def _loss_body(ad_ref, out_ref):
    out_ref[...] = jnp.reshape(jnp.sum(ad_ref[...]) / _B, (1, 1))


def kernel(node_x, edge_attr, edge_index, schedule, y, W_node, b_node,
           W_edge, b_edge, W_pinit, b_pinit, Wa, Wb, Wo, bo, Wnu, bnu,
           W_dec, b_dec):
    f32 = jnp.float32
    src = edge_index[0].astype(jnp.int32)
    dst = edge_index[1].astype(jnp.int32)
    gb = src // _N
    order = jnp.argsort(gb)
    gb_s = gb[order]
    z_s = ((src % _N) * _NP + (dst % _N))[order].astype(jnp.int32)
    starts = jnp.searchsorted(gb_s, jnp.arange(_B + 1, dtype=jnp.int32),
                              side="left").astype(jnp.int32)
    ea_s = edge_attr[order].astype(f32)
    sched_flat = schedule.astype(jnp.int32).reshape(-1)

    ef2, nf0 = pl.pallas_call(
        _prelim_body,
        out_shape=(jax.ShapeDtypeStruct((_E, _D), f32),
                   jax.ShapeDtypeStruct((_B * _N, _D), f32)),
    )(ea_s, node_x.astype(f32), W_edge.astype(f32),
      b_edge.astype(f32).reshape(1, _D), W_pinit.astype(f32),
      b_pinit.astype(f32).reshape(1, _D), W_node.astype(f32),
      b_node.astype(f32).reshape(1, _D))

    grid_spec = pltpu.PrefetchScalarGridSpec(
        num_scalar_prefetch=3,
        grid=(_B,),
        in_specs=[
            pl.BlockSpec((_E, _D), lambda b, *_: (0, 0)),
            pl.BlockSpec((1, _N, _D), lambda b, *_: (b, 0, 0)),
            pl.BlockSpec((1, 1, 1), lambda b, *_: (b, 0, 0)),
            pl.BlockSpec((_L, _S, _D), lambda b, *_: (0, 0, 0)),
            pl.BlockSpec((_L, _S, _D), lambda b, *_: (0, 0, 0)),
            pl.BlockSpec((_L, _D, _S), lambda b, *_: (0, 0, 0)),
            pl.BlockSpec((_L, _S, _D), lambda b, *_: (0, 0, 0)),
            pl.BlockSpec((_L, _D, 1), lambda b, *_: (0, 0, 0)),
            pl.BlockSpec((_L, 1, _D), lambda b, *_: (0, 0, 0)),
            pl.BlockSpec((_L, _D, _D), lambda b, *_: (0, 0, 0)),
            pl.BlockSpec((_L, 1, _D), lambda b, *_: (0, 0, 0)),
            pl.BlockSpec((_ND, _D), lambda b, *_: (0, 0)),
            pl.BlockSpec((1, _ND), lambda b, *_: (0, 0)),
        ],
        out_specs=pl.BlockSpec((1, 1, 1), lambda b, *_: (b, 0, 0)),
        scratch_shapes=[
            pltpu.VMEM((_N, _NP, _D), f32),
            pltpu.VMEM((_N, _D, _NP), f32),
            pltpu.VMEM((_N, _S, _NP), f32),
            pltpu.VMEM((_S, _NP), f32),
        ],
    )

    absd = pl.pallas_call(
        _main_body,
        grid_spec=grid_spec,
        out_shape=jax.ShapeDtypeStruct((_B, 1, 1), f32),
        compiler_params=pltpu.CompilerParams(
            dimension_semantics=("arbitrary",)),
    )(z_s, starts, sched_flat,
      ef2, nf0.reshape(_B, _N, _D), y.astype(f32).reshape(_B, 1, 1),
      jnp.transpose(Wa.astype(f32), (0, 2, 1)),
      jnp.transpose(Wb.astype(f32), (0, 2, 1)),
      jnp.transpose(Wo.astype(f32), (0, 2, 1)),
      Wo.astype(f32),
      bo.astype(f32).reshape(_L, _D, 1),
      bo.astype(f32).reshape(_L, 1, _D),
      Wnu.astype(f32), bnu.astype(f32).reshape(_L, 1, _D),
      jnp.transpose(W_dec.astype(f32)), b_dec.astype(f32).reshape(1, _ND))

    loss = pl.pallas_call(
        _loss_body,
        out_shape=jax.ShapeDtypeStruct((1, 1), f32),
    )(absd)
    return loss[0, 0]
